# baseline (speedup 1.0000x reference)
import jax
import jax.numpy as jnp
from jax.experimental import pallas as pl
from jax.experimental.pallas import tpu as pltpu

_N = 1000000
_D = 16
_BLOCK = 32768
_GRID = (_N + _BLOCK - 1) // _BLOCK


def _copy_body(u_in, i_in, u_out, i_out):
    u_out[...] = u_in[...]
    i_out[...] = i_in[...]


def kernel(adj, user_emb, item_emb):
    del adj
    ut = user_emb.T
    it = item_emb.T
    spec = pl.BlockSpec((_D, _BLOCK), lambda g: (0, g))
    uo, io = pl.pallas_call(
        _copy_body,
        grid=(_GRID,),
        in_specs=[spec, spec],
        out_specs=(spec, spec),
        out_shape=(
            jax.ShapeDtypeStruct((_D, _N), jnp.float32),
            jax.ShapeDtypeStruct((_D, _N), jnp.float32),
        ),
    )(ut, it)
    return uo.T, io.T


# speedup vs baseline: 1.0335x; 1.0335x over previous
import jax
import jax.numpy as jnp
from jax.experimental import pallas as pl
from jax.experimental.pallas import tpu as pltpu

_N = 1000000
_D = 16
_BLOCK = 98304
_GRID = (_N + _BLOCK - 1) // _BLOCK


def _copy_body(u_in, i_in, u_out, i_out):
    u_out[...] = u_in[...]
    i_out[...] = i_in[...]


def kernel(adj, user_emb, item_emb):
    del adj
    ut = user_emb.T
    it = item_emb.T
    spec = pl.BlockSpec((_D, _BLOCK), lambda g: (0, g))
    uo, io = pl.pallas_call(
        _copy_body,
        grid=(_GRID,),
        in_specs=[spec, spec],
        out_specs=(spec, spec),
        out_shape=(
            jax.ShapeDtypeStruct((_D, _N), jnp.float32),
            jax.ShapeDtypeStruct((_D, _N), jnp.float32),
        ),
    )(ut, it)
    return uo.T, io.T
